# Initial kernel scaffold; baseline (speedup 1.0000x reference)
import jax
import jax.numpy as jnp
from jax.experimental import pallas as pl


def kernel(x, edge_attr, edge_index, params):
    raise NotImplementedError("write your pallas kernel here")


# speedup vs baseline: 3.4735x; 3.4735x over previous
import functools

import jax
import jax.numpy as jnp
from jax import lax
from jax.experimental import pallas as pl
from jax.experimental.pallas import tpu as pltpu
from jax.experimental.pallas import tpu_sc as plsc

_N = 10000
_E = 320000
_L = 128

_RN = 2000
_RE = 4000
_NW = 32
_CH = 80
_PW = _E // _NW
_NCH = _PW // _CH
_NPAD = 10240
_RPT = _NPAD // 16


def _silu(x):
    return x * jax.nn.sigmoid(x)


def _ln(h, g, b):
    mu = jnp.mean(h, axis=-1, keepdims=True)
    var = jnp.mean((h - mu) ** 2, axis=-1, keepdims=True)
    return (h - mu) / jnp.sqrt(var + 1e-5) * g + b


def _dot(a, b):
    return jnp.dot(a, b, preferred_element_type=jnp.float32)


def _row2(v):
    return v.reshape(1, -1)


def _full_spec(shape):
    return pl.BlockSpec(shape, lambda i: (0,) * len(shape))


def _row_spec(r, c):
    return pl.BlockSpec((r, c), lambda i: (i, 0))


def _enc_node_body(x, w0, w1, w2, b0, b1, b2, g, bln, wd, ws, hn, a, b):
    h = _silu(_dot(x[...], w0[...]) + b0[...])
    h = _silu(_dot(h, w1[...]) + b1[...])
    h = _dot(h, w2[...]) + b2[...]
    h = _ln(h, g[...], bln[...])
    hn[...] = h
    a[...] = _dot(h, wd[...])
    b[...] = _dot(h, ws[...])


def _enc_node(x, p, wd, ws):
    f = jnp.float32
    args = (x, p["Ws"][0], p["Ws"][1], p["Ws"][2], _row2(p["bs"][0]),
            _row2(p["bs"][1]), _row2(p["bs"][2]), _row2(p["g"]), _row2(p["b2"]),
            wd, ws)
    in_specs = [_row_spec(_RN, _L)] + [_full_spec(a.shape) for a in args[1:]]
    return pl.pallas_call(
        _enc_node_body,
        grid=(_N // _RN,),
        in_specs=in_specs,
        out_specs=[_row_spec(_RN, _L)] * 3,
        out_shape=[jax.ShapeDtypeStruct((_N, _L), f)] * 3,
    )(*args)


def _enc_edge_body(ea, w0, w1, w2, b0, b1, b2, g, bln, he):
    h = _silu(_dot(ea[...], w0[...]) + b0[...])
    h = _silu(_dot(h, w1[...]) + b1[...])
    h = _dot(h, w2[...]) + b2[...]
    he[...] = _ln(h, g[...], bln[...])


def _enc_edge(ea_pad, w0_pad, p):
    f = jnp.float32
    args = (ea_pad, w0_pad, p["Ws"][1], p["Ws"][2], _row2(p["bs"][0]),
            _row2(p["bs"][1]), _row2(p["bs"][2]), _row2(p["g"]), _row2(p["b2"]))
    in_specs = [_row_spec(_RE, 8)] + [_full_spec(a.shape) for a in args[1:]]
    return pl.pallas_call(
        _enc_edge_body,
        grid=(_E // _RE,),
        in_specs=in_specs,
        out_specs=_row_spec(_RE, _L),
        out_shape=jax.ShapeDtypeStruct((_E, _L), f),
    )(*args)


def _edge_block_body(g1, g2, he, w1e, w2, w3, b1, b2, b3, g, bln, eo, hn):
    t = g1[...] + g2[...] + _dot(he[...], w1e[...]) + b1[...]
    t = _silu(t)
    t = _silu(_dot(t, w2[...]) + b2[...])
    t = _dot(t, w3[...]) + b3[...]
    e = _ln(t, g[...], bln[...])
    eo[...] = e
    hn[...] = he[...] + e


def _edge_block(g1, g2, he, p):
    f = jnp.float32
    w1e = p["Ws"][0][2 * _L:]
    args = (g1, g2, he, w1e, p["Ws"][1], p["Ws"][2], _row2(p["bs"][0]),
            _row2(p["bs"][1]), _row2(p["bs"][2]), _row2(p["g"]), _row2(p["b2"]))
    in_specs = [_row_spec(_RE, _L)] * 3 + [_full_spec(a.shape) for a in args[3:]]
    return pl.pallas_call(
        _edge_block_body,
        grid=(_E // _RE,),
        in_specs=in_specs,
        out_specs=[_row_spec(_RE, _L)] * 2,
        out_shape=[jax.ShapeDtypeStruct((_E, _L), f)] * 2,
    )(*args)


def _node_block_body_ab(hn, p0, p1, wh, wa, w1, w2, b0, b1, b2, g, bln, wd, ws,
                        out, a, b):
    agg = p0[...] + p1[...]
    h = _silu(_dot(hn[...], wh[...]) + _dot(agg, wa[...]) + b0[...])
    h = _silu(_dot(h, w1[...]) + b1[...])
    h = _dot(h, w2[...]) + b2[...]
    h = _ln(h, g[...], bln[...])
    hnew = hn[...] + h
    out[...] = hnew
    a[...] = _dot(hnew, wd[...])
    b[...] = _dot(hnew, ws[...])


def _node_block_body(hn, p0, p1, wh, wa, w1, w2, b0, b1, b2, g, bln, out):
    agg = p0[...] + p1[...]
    h = _silu(_dot(hn[...], wh[...]) + _dot(agg, wa[...]) + b0[...])
    h = _silu(_dot(h, w1[...]) + b1[...])
    h = _dot(h, w2[...]) + b2[...]
    h = _ln(h, g[...], bln[...])
    out[...] = hn[...] + h


def _node_block(hn, p0, p1, p, wd, ws):
    f = jnp.float32
    w0 = p["Ws"][0]
    base = (hn, p0, p1, w0[:_L], w0[_L:], p["Ws"][1], p["Ws"][2],
            _row2(p["bs"][0]), _row2(p["bs"][1]), _row2(p["bs"][2]),
            _row2(p["g"]), _row2(p["b2"]))
    if wd is None:
        args = base
        body, n_out = _node_block_body, 1
    else:
        args = base + (wd, ws)
        body, n_out = _node_block_body_ab, 3
    in_specs = [_row_spec(_RN, _L)] * 3 + [_full_spec(a.shape) for a in args[3:]]
    out = pl.pallas_call(
        body,
        grid=(_N // _RN,),
        in_specs=in_specs,
        out_specs=[_row_spec(_RN, _L)] * n_out,
        out_shape=[jax.ShapeDtypeStruct((_N, _L), f)] * n_out,
    )(*args)
    return out if wd is not None else (out[0], None, None)


def _dec_body(hn, w0, w1, w2, b0, b1, b2, out):
    h = _silu(_dot(hn[...], w0[...]) + b0[...])
    h = _silu(_dot(h, w1[...]) + b1[...])
    out[...] = _dot(h, w2[...]) + b2[...]


def _dec(hn, p, w2_pad, b2_pad):
    f = jnp.float32
    args = (hn, p["Ws"][0], p["Ws"][1], w2_pad, _row2(p["bs"][0]),
            _row2(p["bs"][1]), b2_pad)
    in_specs = [_row_spec(_RN, _L)] + [_full_spec(a.shape) for a in args[1:]]
    return pl.pallas_call(
        _dec_body,
        grid=(_N // _RN,),
        in_specs=in_specs,
        out_specs=_row_spec(_RN, _L),
        out_shape=jax.ShapeDtypeStruct((_N, _L), f),
    )(*args)


def _sc_gather(a_tab, b_tab, dst, src):
    f = jnp.float32
    mesh = plsc.VectorSubcoreMesh(core_axis_name="c", subcore_axis_name="s")

    @functools.partial(
        pl.kernel,
        out_type=[jax.ShapeDtypeStruct((_E, _L), f)] * 2,
        mesh=mesh,
        scratch_types=[
            pltpu.VMEM((_PW,), jnp.int32),
            pltpu.VMEM((_PW,), jnp.int32),
            pltpu.VMEM((_CH, _L), f),
            pltpu.VMEM((_CH, _L), f),
            pltpu.SemaphoreType.DMA,
            pltpu.SemaphoreType.DMA,
        ],
    )
    def run(a_h, b_h, dst_h, src_h, g1_h, g2_h, di, si, ra, rb, s1, s2):
        wid = lax.axis_index("s") * 2 + lax.axis_index("c")
        base = wid * _PW
        pltpu.sync_copy(dst_h.at[pl.ds(base, _PW)], di)
        pltpu.sync_copy(src_h.at[pl.ds(base, _PW)], si)

        def body(i, carry):
            off = i * _CH
            ca = pltpu.async_copy(a_h.at[di.at[pl.ds(off, _CH)]], ra, s1)
            cb = pltpu.async_copy(b_h.at[si.at[pl.ds(off, _CH)]], rb, s2)
            ca.wait()
            cb.wait()
            pltpu.sync_copy(ra, g1_h.at[pl.ds(base + off, _CH)])
            pltpu.sync_copy(rb, g2_h.at[pl.ds(base + off, _CH)])
            return carry

        lax.fori_loop(0, _NCH, body, 0)

    return run(a_tab, b_tab, dst, src)


def _sc_scatter(e_out, dst, zeros_blk):
    f = jnp.float32
    mesh = plsc.VectorSubcoreMesh(core_axis_name="c", subcore_axis_name="s")

    @functools.partial(
        pl.kernel,
        out_type=jax.ShapeDtypeStruct((2, _NPAD, _L), f),
        mesh=mesh,
        scratch_types=[
            pltpu.VMEM((_CH,), jnp.int32),
            pltpu.VMEM((_CH, _L), f),
            pltpu.VMEM_SHARED((_NPAD, _L), f),
        ],
    )
    def run(e_h, dst_h, z_h, out_h, di, rows, acc):
        c = lax.axis_index("c")
        s = lax.axis_index("s")
        wid = s * 2 + c
        pltpu.sync_copy(z_h, rows)
        for j in range(_RPT // _CH):
            pltpu.sync_copy(rows, acc.at[pl.ds(s * _RPT + j * _CH, _CH)])
        plsc.subcore_barrier()

        def body(i, carry):
            off = wid * _PW + i * _CH
            pltpu.sync_copy(dst_h.at[pl.ds(off, _CH)], di)
            pltpu.sync_copy(e_h.at[pl.ds(off, _CH)], rows)
            pltpu.sync_copy(rows, acc.at[di], add=True)
            return carry

        lax.fori_loop(0, _NCH, body, 0)
        plsc.subcore_barrier()
        pltpu.sync_copy(acc.at[pl.ds(s * _RPT, _RPT)],
                        out_h.at[c, pl.ds(s * _RPT, _RPT)])

    return run(e_out, dst, zeros_blk)[:, :_N, :]


def kernel(x, edge_attr, edge_index, params):
    f = jnp.float32
    src = edge_index[0]
    dst = edge_index[1]
    blocks = params["blocks"]

    w1_0 = blocks[0]["eb"]["Ws"][0]
    h_n, a_tab, b_tab = _enc_node(x, params["node_enc"],
                                  w1_0[:_L], w1_0[_L:2 * _L])

    ea_pad = jnp.pad(edge_attr, ((0, 0), (0, 4)))
    ew0 = params["edge_enc"]["Ws"][0]
    ew0_pad = jnp.pad(ew0, ((0, 8 - ew0.shape[0]), (0, 0)))
    h_e = _enc_edge(ea_pad, ew0_pad, params["edge_enc"])

    zeros_blk = jnp.zeros((_CH, _L), f)

    for bi, blk in enumerate(blocks):
        g1, g2 = _sc_gather(a_tab, b_tab, dst, src)
        e_out, h_e = _edge_block(g1, g2, h_e, blk["eb"])
        parts = _sc_scatter(e_out, dst, zeros_blk)
        if bi + 1 < len(blocks):
            w1_n = blocks[bi + 1]["eb"]["Ws"][0]
            h_n, a_tab, b_tab = _node_block(h_n, parts[0], parts[1],
                                            blk["nb"], w1_n[:_L],
                                            w1_n[_L:2 * _L])
        else:
            h_n, _, _ = _node_block(h_n, parts[0], parts[1], blk["nb"],
                                    None, None)

    dp = params["dec"]
    w2_pad = jnp.pad(dp["Ws"][2], ((0, 0), (0, _L - dp["Ws"][2].shape[1])))
    b2_pad = jnp.pad(_row2(dp["bs"][2]), ((0, 0), (0, _L - dp["bs"][2].shape[0])))
    out = _dec(h_n, dp, w2_pad, b2_pad)
    return out[:, :dp["Ws"][2].shape[1]]
